# Initial kernel scaffold; baseline (speedup 1.0000x reference)
import jax
import jax.numpy as jnp
from jax.experimental import pallas as pl


def kernel(x, table):
    raise NotImplementedError("write your pallas kernel here")


# speedup vs baseline: 9.3080x; 9.3080x over previous
import functools

import jax
import jax.numpy as jnp
from jax import lax
from jax.experimental import pallas as pl
from jax.experimental.pallas import tpu as pltpu
from jax.experimental.pallas import tpu_sc as plsc

_L = 16


def _make_sc_gather(n_total, n_workers, chunk, d):
    per_worker = n_total // n_workers
    n_chunks = per_worker // chunk
    mesh = plsc.VectorSubcoreMesh(core_axis_name="c", subcore_axis_name="s")

    @functools.partial(
        pl.kernel,
        mesh=mesh,
        out_type=jax.ShapeDtypeStruct((n_total * d,), jnp.float32),
        scratch_types=[
            pltpu.VMEM((32,), jnp.float32),
            pltpu.VMEM((chunk,), jnp.int32),
            pltpu.VMEM((chunk * d,), jnp.float32),
        ],
        compiler_params=pltpu.CompilerParams(needs_layout_passes=False),
    )
    def k(tab_hbm, idx_hbm, out_hbm, tab_v, idx_v, out_v):
        nc = 2
        wid = lax.axis_index("s") * nc + lax.axis_index("c")
        pltpu.sync_copy(tab_hbm, tab_v)
        lane = lax.iota(jnp.int32, _L)
        lane4 = lane * 4

        def do_chunk(c, _):
            start = wid * per_worker + c * chunk
            pltpu.sync_copy(idx_hbm.at[pl.ds(start, chunk)], idx_v)

            def body(i, _):
                base = i * _L
                xi = idx_v[pl.ds(base, _L)]
                sbase = lane4 + base * 4
                for j in range(d):
                    col = plsc.load_gather(tab_v, [xi + (8 * j)])
                    plsc.store_scatter(out_v, [sbase + j], col)
                return 0

            lax.fori_loop(0, chunk // _L, body, 0, unroll=4)
            pltpu.sync_copy(out_v, out_hbm.at[pl.ds(start * d, chunk * d)])
            return 0

        lax.fori_loop(0, n_chunks, do_chunk, 0)

    return k


def kernel(x, table):
    b, s = x.shape
    v, d = table.shape
    n = b * s
    x_flat = x.reshape(n).astype(jnp.int32)
    tab_cm = table.T.reshape(v * d)
    n_workers = 32
    chunk = 6400
    out_flat = _make_sc_gather(n, n_workers, chunk, d)(tab_cm, x_flat)
    return out_flat.reshape(b, s, d)


# speedup vs baseline: 10.1825x; 1.0940x over previous
import functools

import jax
import jax.numpy as jnp
from jax import lax
from jax.experimental import pallas as pl
from jax.experimental.pallas import tpu as pltpu
from jax.experimental.pallas import tpu_sc as plsc

_L = 16


def _make_sc_gather(n_total, n_workers, chunk, d):
    per_worker = n_total // n_workers
    n_chunks = per_worker // chunk
    mesh = plsc.VectorSubcoreMesh(core_axis_name="c", subcore_axis_name="s")

    @functools.partial(
        pl.kernel,
        mesh=mesh,
        out_type=jax.ShapeDtypeStruct((n_total * d,), jnp.float32),
        scratch_types=[
            pltpu.VMEM((32,), jnp.float32),
            pltpu.VMEM((chunk,), jnp.int32),
            pltpu.VMEM((chunk,), jnp.int32),
            pltpu.VMEM((chunk * d,), jnp.float32),
            pltpu.VMEM((chunk * d,), jnp.float32),
            pltpu.SemaphoreType.DMA,
            pltpu.SemaphoreType.DMA,
            pltpu.SemaphoreType.DMA,
            pltpu.SemaphoreType.DMA,
        ],
        compiler_params=pltpu.CompilerParams(needs_layout_passes=False),
    )
    def k(tab_hbm, idx_hbm, out_hbm, tab_v, ib0, ib1, ob0, ob1, si0, si1, so0, so1):
        nc = 2
        wid = lax.axis_index("s") * nc + lax.axis_index("c")
        base0 = wid * per_worker
        pltpu.sync_copy(tab_hbm, tab_v)
        lane = lax.iota(jnp.int32, _L)
        lane4 = lane * 4
        sem_in = [si0, si1]
        sem_out = [so0, so1]
        idx_bufs = [ib0, ib1]
        out_bufs = [ob0, ob1]

        def start_in(c):
            return pltpu.async_copy(
                idx_hbm.at[pl.ds(base0 + c * chunk, chunk)],
                idx_bufs[c % 2],
                sem_in[c % 2],
            )

        def start_out(c):
            return pltpu.async_copy(
                out_bufs[c % 2],
                out_hbm.at[pl.ds((base0 + c * chunk) * d, chunk * d)],
                sem_out[c % 2],
            )

        cp_in = {0: start_in(0)}
        cp_out = {}
        for c in range(n_chunks):
            b = c % 2
            if c + 1 < n_chunks:
                cp_in[c + 1] = start_in(c + 1)
            cp_in[c].wait()
            if c >= 2:
                cp_out[c - 2].wait()
            idx_c = idx_bufs[b]
            out_c = out_bufs[b]

            @plsc.parallel_loop(0, chunk, _L, unroll=4)
            def body(base):
                xi = idx_c[pl.ds(base, _L)]
                sbase = lane4 + base * 4
                for j in range(d):
                    col = plsc.load_gather(tab_v, [xi + (8 * j)])
                    plsc.store_scatter(out_c, [sbase + j], col)

            cp_out[c] = start_out(c)
        cp_out[n_chunks - 2].wait()
        cp_out[n_chunks - 1].wait()

    return k


def kernel(x, table):
    b, s = x.shape
    v, d = table.shape
    n = b * s
    x_flat = x.reshape(n).astype(jnp.int32)
    tab_cm = table.T.reshape(v * d)
    n_workers = 32
    chunk = 6400
    out_flat = _make_sc_gather(n, n_workers, chunk, d)(tab_cm, x_flat)
    return out_flat.reshape(b, s, d)
